# baseline (speedup 1.0000x reference)
import functools

import jax
import jax.numpy as jnp
from jax import lax
from jax.experimental import pallas as pl
from jax.experimental.pallas import tpu as pltpu
from jax.experimental.pallas import tpu_sc as plsc

EMBED_DIM = 300
PAD_DIM = 304
NUM_CORES = 2
NUM_SUBCORES = 16
NUM_WORKERS = NUM_CORES * NUM_SUBCORES
CHUNK = 128


def _make_gather(batch: int):
  assert batch % (NUM_WORKERS * CHUNK) == 0
  bpw = batch // NUM_WORKERS
  nchunk = bpw // CHUNK

  mesh = plsc.VectorSubcoreMesh(core_axis_name="c", subcore_axis_name="s")

  @functools.partial(
      pl.kernel,
      mesh=mesh,
      out_type=jax.ShapeDtypeStruct((batch, PAD_DIM), jnp.float32),
      scratch_types=[
          pltpu.VMEM((CHUNK,), jnp.int32),
          pltpu.VMEM((CHUNK, PAD_DIM), jnp.float32),
          pltpu.SemaphoreType.DMA,
      ],
      compiler_params=pltpu.CompilerParams(use_tc_tiling_on_sc=False),
  )
  def gather_kernel(x_hbm, table_hbm, out_hbm, idx_v, rows_v, sem):
    wid = lax.axis_index("s") * NUM_CORES + lax.axis_index("c")
    base = wid * bpw

    def body(i, carry):
      off = base + i * CHUNK
      pltpu.sync_copy(x_hbm.at[pl.ds(off, CHUNK)], idx_v)
      pltpu.async_copy(table_hbm.at[idx_v], rows_v, sem).wait()
      pltpu.sync_copy(rows_v, out_hbm.at[pl.ds(off, CHUNK)])
      return carry

    lax.fori_loop(0, nchunk, body, 0)

  return gather_kernel


def kernel(x, table):
  b0, b1 = x.shape
  xf = x.reshape(b0 * b1)
  table_p = jnp.pad(table, ((0, 0), (0, PAD_DIM - EMBED_DIM)))
  out = _make_gather(b0 * b1)(xf, table_p)
  return out[:, :EMBED_DIM].reshape(b0, b1, EMBED_DIM)
